# baseline (speedup 1.0000x reference)
import jax
import jax.numpy as jnp
from jax.experimental import pallas as pl


_BB = 32


def _add_body(x_ref, t_ref, o_ref):
    o_ref[...] = x_ref[...] + t_ref[...][None]


def kernel(x, pos_table):
    B, L, D = x.shape
    return pl.pallas_call(
        _add_body,
        grid=(B // _BB,),
        in_specs=[
            pl.BlockSpec((_BB, L, D), lambda i: (i, 0, 0)),
            pl.BlockSpec((L, D), lambda i: (0, 0)),
        ],
        out_specs=pl.BlockSpec((_BB, L, D), lambda i: (i, 0, 0)),
        out_shape=jax.ShapeDtypeStruct((B, L, D), x.dtype),
    )(x, pos_table)


# speedup vs baseline: 1.0028x; 1.0028x over previous
import jax
import jax.numpy as jnp
from jax.experimental import pallas as pl
from jax.experimental.pallas import tpu as pltpu


_BB = 64


def _add_body(x_ref, t_ref, o_ref):
    o_ref[...] = x_ref[...] + t_ref[...][None]


def kernel(x, pos_table):
    B, L, D = x.shape
    return pl.pallas_call(
        _add_body,
        grid=(B // _BB,),
        in_specs=[
            pl.BlockSpec((_BB, L, D), lambda i: (i, 0, 0)),
            pl.BlockSpec((L, D), lambda i: (0, 0)),
        ],
        out_specs=pl.BlockSpec((_BB, L, D), lambda i: (i, 0, 0)),
        out_shape=jax.ShapeDtypeStruct((B, L, D), x.dtype),
        compiler_params=pltpu.CompilerParams(
            dimension_semantics=("parallel",),
        ),
    )(x, pos_table)


# speedup vs baseline: 6.0596x; 6.0425x over previous
import jax
import jax.numpy as jnp
from jax.experimental import pallas as pl
from jax.experimental.pallas import tpu as pltpu


_LANES = 128


def _add_body(x_ref, t_ref, o_ref):
    o_ref[...] = x_ref[...] + t_ref[...]


def kernel(x, pos_table):
    B, L, D = x.shape
    xt = x.transpose(1, 2, 0)
    tb = jnp.broadcast_to(pos_table[:, :, None], (L, D, _LANES))
    out_t = pl.pallas_call(
        _add_body,
        grid=(B // _LANES,),
        in_specs=[
            pl.BlockSpec((L, D, _LANES), lambda i: (0, 0, i)),
            pl.BlockSpec((L, D, _LANES), lambda i: (0, 0, 0)),
        ],
        out_specs=pl.BlockSpec((L, D, _LANES), lambda i: (0, 0, i)),
        out_shape=jax.ShapeDtypeStruct((L, D, B), x.dtype),
        compiler_params=pltpu.CompilerParams(
            dimension_semantics=("arbitrary",),
        ),
    )(xt, tb)
    return out_t.transpose(2, 0, 1)


# speedup vs baseline: 6.2214x; 1.0267x over previous
import jax
import jax.numpy as jnp
from jax.experimental import pallas as pl
from jax.experimental.pallas import tpu as pltpu


_LANES = 128


def _add_body(x_ref, t_ref, o_ref, tb_ref):
    @pl.when(pl.program_id(0) == 0)
    def _():
        tb_ref[...] = jax.lax.broadcast_in_dim(
            t_ref[...], tb_ref.shape, (0, 1)
        )

    o_ref[...] = x_ref[...] + tb_ref[...]


def kernel(x, pos_table):
    B, L, D = x.shape
    xt = x.transpose(1, 2, 0)
    out_t = pl.pallas_call(
        _add_body,
        grid=(B // _LANES,),
        in_specs=[
            pl.BlockSpec((L, D, _LANES), lambda i: (0, 0, i)),
            pl.BlockSpec((L, D), lambda i: (0, 0)),
        ],
        out_specs=pl.BlockSpec((L, D, _LANES), lambda i: (0, 0, i)),
        out_shape=jax.ShapeDtypeStruct((L, D, B), x.dtype),
        scratch_shapes=[pltpu.VMEM((L, D, _LANES), x.dtype)],
        compiler_params=pltpu.CompilerParams(
            dimension_semantics=("arbitrary",),
        ),
    )(xt, pos_table)
    return out_t.transpose(2, 0, 1)
